# baseline (speedup 1.0000x reference)
import functools

import jax
import jax.numpy as jnp
from jax import lax
from jax.experimental import pallas as pl
from jax.experimental.pallas import tpu as pltpu
from jax.experimental.pallas import tpu_sc as plsc

N = 100000
E = 1600000
H = 32
HH = 16
NC = 2
NS = 16

CHUNK = 512
RPC = CHUNK // 128
NCHUNK = 200
EPT = CHUNK * NCHUNK
E_PAD = EPT * NS
EROWS = E_PAD // 128
PAD = E_PAD - E
NPADROW = 96
R = N + NPADROW
ZPT = R // NS
WB = 6256
WB_LAST = N - 15 * WB

_MESH = dict(core_axis_name="c", subcore_axis_name="s", num_cores=NC,
             num_subcores=NS)
_SC_PARAMS = pltpu.CompilerParams(use_tc_tiling_on_sc=False)


def _zero_rows(rows_v, nrows):
    zero = jnp.zeros((16,), jnp.float32)

    def body(i, _):
        rows_v[i, :] = zero
        return ()

    lax.fori_loop(0, nrows, body, ())


def _make_seg_kernel(with_counts: bool):
    nseg = jax.ShapeDtypeStruct((N, HH), jnp.float32)
    ncnt = jax.ShapeDtypeStruct((N,), jnp.float32)
    out_type = [nseg, nseg, nseg, nseg]
    scratch = [
        pltpu.VMEM((2, RPC, 128), jnp.int32),
        pltpu.VMEM((2, RPC, 128), jnp.int32),
        pltpu.VMEM((2, CHUNK, HH), jnp.float32),
        pltpu.VMEM_SHARED((R, HH), jnp.float32),
        pltpu.SemaphoreType.DMA,
        pltpu.SemaphoreType.DMA,
        pltpu.SemaphoreType.DMA,
    ]
    if with_counts:
        out_type += [ncnt, ncnt, ncnt, ncnt]
        scratch += [
            pltpu.VMEM_SHARED((R,), jnp.float32),
            pltpu.VMEM((3128,), jnp.float32),
            pltpu.VMEM((128,), jnp.float32),
        ]

    @functools.partial(
        pl.kernel, out_type=out_type,
        mesh=plsc.VectorSubcoreMesh(**_MESH),
        scratch_types=scratch, compiler_params=_SC_PARAMS)
    def seg_kernel(src_r, dst_r, yr_lo, yr_hi, src_b, dst_b, yb_lo, yb_hi,
                   *rest):
        if with_counts:
            (o_r_lo, o_r_hi, o_b_lo, o_b_hi, cm0, cm1, cu0, cu1,
             src_v, dst_v, rows_v, acc, gsem, ssem, psem,
             cacc, zcb, ones_v) = rest
        else:
            (o_r_lo, o_r_hi, o_b_lo, o_b_hi,
             src_v, dst_v, rows_v, acc, gsem, ssem, psem) = rest
            cacc = zcb = ones_v = None
        c = lax.axis_index("c")
        t = lax.axis_index("s")

        if with_counts:
            for j in range(3128 // 16):
                zcb[pl.ds(j * 16, 16)] = jnp.zeros((16,), jnp.float32)
            for j in range(8):
                ones_v[pl.ds(j * 16, 16)] = jnp.ones((16,), jnp.float32)

        def zero_acc():
            _zero_rows(rows_v.at[0], CHUNK)
            base = t * ZPT
            for j in range(ZPT // CHUNK):
                pltpu.sync_copy(rows_v.at[0],
                                acc.at[pl.ds(base + j * CHUNK, CHUNK), :])
            rem = ZPT % CHUNK
            pltpu.sync_copy(rows_v.at[0, pl.ds(0, rem), :],
                            acc.at[pl.ds(base + (ZPT // CHUNK) * CHUNK,
                                         rem), :])
            if with_counts:
                pltpu.sync_copy(zcb, cacc.at[pl.ds(base, 3128)])
                pltpu.sync_copy(zcb, cacc.at[pl.ds(base + 3128, 3128)])

        def count_cond(ci):
            return jnp.equal(ci < NCHUNK // 2, c == 0)

        def run_pipeline(src_hbm, dst_hbm, y_hbm):
            def idx_row0(ci):
                return t * (NCHUNK * RPC) + ci * RPC

            def fire_idx(b, ci, sync=False):
                copy = pltpu.sync_copy if sync else (
                    lambda s, d: pltpu.async_copy(s, d, psem))
                copy(src_hbm.at[pl.ds(idx_row0(ci), RPC), :], src_v.at[b])
                copy(dst_hbm.at[pl.ds(idx_row0(ci), RPC), :], dst_v.at[b])

            def wait_idx(b, ci):
                pltpu.make_async_copy(
                    src_hbm.at[pl.ds(idx_row0(ci), RPC), :],
                    src_v.at[b], psem).wait()
                pltpu.make_async_copy(
                    dst_hbm.at[pl.ds(idx_row0(ci), RPC), :],
                    dst_v.at[b], psem).wait()

            def fire_gathers(b):
                for j in range(RPC):
                    pltpu.async_copy(y_hbm.at[src_v.at[b, j]],
                                     rows_v.at[b, pl.ds(j * 128, 128), :],
                                     gsem)

            def wait_gathers(b):
                for j in range(RPC):
                    pltpu.make_async_copy(
                        y_hbm.at[src_v.at[b, j]],
                        rows_v.at[b, pl.ds(j * 128, 128), :], gsem).wait()

            def fire_scatters(b, ci):
                for j in range(RPC):
                    pltpu.async_copy(rows_v.at[b, pl.ds(j * 128, 128), :],
                                     acc.at[dst_v.at[b, j]], ssem, add=True)
                if with_counts:
                    @pl.when(count_cond(ci))
                    def _():
                        for j in range(RPC):
                            pltpu.async_copy(ones_v, cacc.at[dst_v.at[b, j]],
                                             ssem, add=True)

            def wait_scatters(b, ci):
                for j in range(RPC):
                    pltpu.make_async_copy(rows_v.at[b, pl.ds(j * 128, 128), :],
                                          acc.at[dst_v.at[b, j]], ssem).wait()
                if with_counts:
                    @pl.when(count_cond(ci))
                    def _():
                        for j in range(RPC):
                            pltpu.make_async_copy(
                                ones_v, cacc.at[dst_v.at[b, j]], ssem).wait()

            fire_idx(0, 0, sync=True)
            fire_gathers(0)

            def pair_body(g, _):
                for b in (0, 1):
                    ci = 2 * g + b

                    @pl.when(ci >= 1)
                    def _():
                        wait_scatters(1 - b, ci - 1)

                    @pl.when(ci <= NCHUNK - 2)
                    def _():
                        fire_idx(1 - b, ci + 1)

                    wait_gathers(b)
                    fire_scatters(b, ci)

                    @pl.when(ci <= NCHUNK - 2)
                    def _():
                        wait_idx(1 - b, ci + 1)
                        fire_gathers(1 - b)
                return ()

            lax.fori_loop(0, NCHUNK // 2, pair_body, ())
            wait_scatters(1, NCHUNK - 1)

        def writeback(out_lo, out_hi, cnt_lo, cnt_hi):
            def wb_one(out_ref, cnt_ref):
                @pl.when(t < NS - 1)
                def _():
                    pltpu.sync_copy(acc.at[pl.ds(t * WB, WB), :],
                                    out_ref.at[pl.ds(t * WB, WB), :])
                    if with_counts:
                        pltpu.sync_copy(cacc.at[pl.ds(t * WB, WB)],
                                        cnt_ref.at[pl.ds(t * WB, WB)])

                @pl.when(t == NS - 1)
                def _():
                    pltpu.sync_copy(
                        acc.at[pl.ds((NS - 1) * WB, WB_LAST), :],
                        out_ref.at[pl.ds((NS - 1) * WB, WB_LAST), :])
                    if with_counts:
                        pltpu.sync_copy(
                            cacc.at[pl.ds((NS - 1) * WB, WB_LAST)],
                            cnt_ref.at[pl.ds((NS - 1) * WB, WB_LAST)])

            @pl.when(c == 0)
            def _():
                wb_one(out_lo, cnt_lo)

            @pl.when(c == 1)
            def _():
                wb_one(out_hi, cnt_hi)

        def relation(src_hbm, dst_hbm, y_lo, y_hi, out_lo, out_hi,
                     cnt_lo, cnt_hi):
            zero_acc()
            plsc.subcore_barrier()

            @pl.when(c == 0)
            def _():
                run_pipeline(src_hbm, dst_hbm, y_lo)

            @pl.when(c == 1)
            def _():
                run_pipeline(src_hbm, dst_hbm, y_hi)

            plsc.subcore_barrier()
            writeback(out_lo, out_hi, cnt_lo, cnt_hi)
            plsc.subcore_barrier()

        if with_counts:
            relation(src_r, dst_r, yr_lo, yr_hi, o_r_lo, o_r_hi, cm0, cm1)
            relation(src_b, dst_b, yb_lo, yb_hi, o_b_lo, o_b_hi, cu0, cu1)
        else:
            relation(src_r, dst_r, yr_lo, yr_hi, o_r_lo, o_r_hi, None, None)
            relation(src_b, dst_b, yb_lo, yb_hi, o_b_lo, o_b_hi, None, None)

    return seg_kernel


_seg1_kernel = _make_seg_kernel(True)
_seg2_kernel = _make_seg_kernel(False)


def _dotT(x, w):
    return lax.dot_general(x, w, (((1,), (1,)), ((), ())),
                           preferred_element_type=jnp.float32)


BR = 2000


def _tc0_body(xu, xm, wl1m, wr1m, b1m, wl1u, wr1u, b1u,
              yu_lo, yu_hi, ym_lo, ym_hi, rm, ru):
    yu = _dotT(xu[...], wl1m[...])
    yu_lo[...] = yu[:, :HH]
    yu_hi[...] = yu[:, HH:]
    ym = _dotT(xm[...], wl1u[...])
    ym_lo[...] = ym[:, :HH]
    ym_hi[...] = ym[:, HH:]
    rm[...] = _dotT(xm[...], wr1m[...]) + b1m[...]
    ru[...] = _dotT(xu[...], wr1u[...]) + b1u[...]


def _tc2_body(s1m_lo, s1m_hi, cm0, cm1, rm, s1u_lo, s1u_hi, cu0, cu1, ru,
              wl2m, wr2m, b2m, wl2u, wr2u, b2u,
              zu_lo, zu_hi, zm_lo, zm_hi, r2m, r2u):
    inv_cm = 1.0 / jnp.clip(cm0[...] + cm1[...], 1.0)
    inv_cu = 1.0 / jnp.clip(cu0[...] + cu1[...], 1.0)
    m1 = jnp.concatenate([s1m_lo[...], s1m_hi[...]], axis=1) * inv_cm + rm[...]
    u1 = jnp.concatenate([s1u_lo[...], s1u_hi[...]], axis=1) * inv_cu + ru[...]
    zu = _dotT(u1, wl2m[...])
    zu_lo[...] = zu[:, :HH]
    zu_hi[...] = zu[:, HH:]
    zm = _dotT(m1, wl2u[...])
    zm_lo[...] = zm[:, :HH]
    zm_hi[...] = zm[:, HH:]
    r2m[...] = _dotT(m1, wr2m[...]) + b2m[...]
    r2u[...] = _dotT(u1, wr2u[...]) + b2u[...]


def _tc4_body(s2m_lo, s2m_hi, cm0, cm1, r2m, s2u_lo, s2u_hi, cu0, cu1, r2u,
              u2, m2):
    inv_cm = 1.0 / jnp.clip(cm0[...] + cm1[...], 1.0)
    inv_cu = 1.0 / jnp.clip(cu0[...] + cu1[...], 1.0)
    m2[...] = (jnp.concatenate([s2m_lo[...], s2m_hi[...]], axis=1)
               * inv_cm + r2m[...])
    u2[...] = (jnp.concatenate([s2u_lo[...], s2u_hi[...]], axis=1)
               * inv_cu + r2u[...])


def _rows(nc):
    return pl.BlockSpec((BR, nc), lambda i: (i, 0))


def _full(shape):
    nd = len(shape)
    return pl.BlockSpec(shape, lambda i: (0,) * nd)


def _shape(r, c):
    return jax.ShapeDtypeStruct((r, c), jnp.float32)


def kernel(x_user, x_movie, edge_index_rates, edge_index_rated_by,
           Wl1m, b1m, Wr1m, Wl1u, b1u, Wr1u,
           Wl2m, b2m, Wr2m, Wl2u, b2u, Wr2u):
    grid = (N // BR,)

    yu_lo, yu_hi, ym_lo, ym_hi, rm, ru = pl.pallas_call(
        _tc0_body,
        grid=grid,
        in_specs=[_rows(24), _rows(18),
                  _full((H, 24)), _full((H, 18)), _full((1, H)),
                  _full((H, 18)), _full((H, 24)), _full((1, H))],
        out_specs=[_rows(HH), _rows(HH), _rows(HH), _rows(HH),
                   _rows(H), _rows(H)],
        out_shape=[_shape(N, HH), _shape(N, HH), _shape(N, HH), _shape(N, HH),
                   _shape(N, H), _shape(N, H)],
    )(x_user, x_movie, Wl1m, Wr1m, b1m.reshape(1, H),
      Wl1u, Wr1u, b1u.reshape(1, H))

    pad = (jnp.arange(PAD, dtype=jnp.int32) % NPADROW)

    def prep(ei):
        src = jnp.concatenate([ei[0].astype(jnp.int32), pad])
        dst = jnp.concatenate([ei[1].astype(jnp.int32), pad + N])
        return src.reshape(EROWS, 128), dst.reshape(EROWS, 128)

    src_r, dst_r = prep(edge_index_rates)
    src_b, dst_b = prep(edge_index_rated_by)

    (s1m_lo, s1m_hi, s1u_lo, s1u_hi,
     cm0, cm1, cu0, cu1) = _seg1_kernel(src_r, dst_r, yu_lo, yu_hi,
                                        src_b, dst_b, ym_lo, ym_hi)
    cm0, cm1 = cm0.reshape(N, 1), cm1.reshape(N, 1)
    cu0, cu1 = cu0.reshape(N, 1), cu1.reshape(N, 1)

    zu_lo, zu_hi, zm_lo, zm_hi, r2m, r2u = pl.pallas_call(
        _tc2_body,
        grid=grid,
        in_specs=[_rows(HH), _rows(HH), _rows(1), _rows(1), _rows(H),
                  _rows(HH), _rows(HH), _rows(1), _rows(1), _rows(H),
                  _full((H, H)), _full((H, H)), _full((1, H)),
                  _full((H, H)), _full((H, H)), _full((1, H))],
        out_specs=[_rows(HH), _rows(HH), _rows(HH), _rows(HH),
                   _rows(H), _rows(H)],
        out_shape=[_shape(N, HH), _shape(N, HH), _shape(N, HH), _shape(N, HH),
                   _shape(N, H), _shape(N, H)],
    )(s1m_lo, s1m_hi, cm0, cm1, rm, s1u_lo, s1u_hi, cu0, cu1, ru,
      Wl2m, Wr2m, b2m.reshape(1, H), Wl2u, Wr2u, b2u.reshape(1, H))

    s2m_lo, s2m_hi, s2u_lo, s2u_hi = _seg2_kernel(
        src_r, dst_r, zu_lo, zu_hi, src_b, dst_b, zm_lo, zm_hi)

    u2, m2 = pl.pallas_call(
        _tc4_body,
        grid=grid,
        in_specs=[_rows(HH), _rows(HH), _rows(1), _rows(1), _rows(H),
                  _rows(HH), _rows(HH), _rows(1), _rows(1), _rows(H)],
        out_specs=[_rows(H), _rows(H)],
        out_shape=[_shape(N, H), _shape(N, H)],
    )(s2m_lo, s2m_hi, cm0, cm1, r2m, s2u_lo, s2u_hi, cu0, cu1, r2u)

    return (u2, m2)


# speedup vs baseline: 1.0772x; 1.0772x over previous
import functools

import jax
import jax.numpy as jnp
from jax import lax
from jax.experimental import pallas as pl
from jax.experimental.pallas import tpu as pltpu
from jax.experimental.pallas import tpu_sc as plsc

N = 100000
E = 1600000
H = 32
HH = 16
NC = 2
NS = 16

CHUNK = 512
RPC = CHUNK // 128
NCHUNK = 200
EPT = CHUNK * NCHUNK
E_PAD = EPT * NS
EROWS = E_PAD // 128
PAD = E_PAD - E
NPADROW = 96
R = N + NPADROW
ZPT = R // NS
WB = 6256
WB_LAST = N - 15 * WB

_MESH = dict(core_axis_name="c", subcore_axis_name="s", num_cores=NC,
             num_subcores=NS)
_SC_PARAMS = pltpu.CompilerParams(use_tc_tiling_on_sc=False)


def _zero_rows(rows_v, nrows):
    zero = jnp.zeros((16,), jnp.float32)

    def body(i, _):
        rows_v[i, :] = zero
        return ()

    lax.fori_loop(0, nrows, body, ())


def _make_seg_kernel(with_counts: bool):
    nseg = jax.ShapeDtypeStruct((N, HH), jnp.float32)
    ncnt = jax.ShapeDtypeStruct((N,), jnp.float32)
    out_type = [nseg, nseg]
    scratch = [
        pltpu.VMEM((2, RPC, 128), jnp.int32),
        pltpu.VMEM((2, RPC, 128), jnp.int32),
        pltpu.VMEM((2, CHUNK, HH), jnp.float32),
        pltpu.VMEM_SHARED((R, HH), jnp.float32),
        pltpu.SemaphoreType.DMA,
        pltpu.SemaphoreType.DMA,
        pltpu.SemaphoreType.DMA,
    ]
    if with_counts:
        out_type += [ncnt, ncnt]
        scratch += [
            pltpu.VMEM_SHARED((R,), jnp.float32),
            pltpu.VMEM((3128,), jnp.float32),
            pltpu.VMEM((128,), jnp.float32),
        ]

    @functools.partial(
        pl.kernel, out_type=out_type,
        mesh=plsc.VectorSubcoreMesh(**_MESH),
        scratch_types=scratch, compiler_params=_SC_PARAMS)
    def seg_kernel(src_hbm, dst_hbm, y_lo, y_hi, *rest):
        if with_counts:
            (out_lo, out_hi, cnt_lo, cnt_hi,
             src_v, dst_v, rows_v, acc, gsem, ssem, psem,
             cacc, zcb, ones_v) = rest
        else:
            (out_lo, out_hi,
             src_v, dst_v, rows_v, acc, gsem, ssem, psem) = rest
            cacc = zcb = ones_v = cnt_lo = cnt_hi = None
        c = lax.axis_index("c")
        t = lax.axis_index("s")

        if with_counts:
            for j in range(3128 // 16):
                zcb[pl.ds(j * 16, 16)] = jnp.zeros((16,), jnp.float32)
            for j in range(8):
                ones_v[pl.ds(j * 16, 16)] = jnp.ones((16,), jnp.float32)

        def zero_acc():
            _zero_rows(rows_v.at[0], CHUNK)
            base = t * ZPT
            for j in range(ZPT // CHUNK):
                pltpu.sync_copy(rows_v.at[0],
                                acc.at[pl.ds(base + j * CHUNK, CHUNK), :])
            rem = ZPT % CHUNK
            pltpu.sync_copy(rows_v.at[0, pl.ds(0, rem), :],
                            acc.at[pl.ds(base + (ZPT // CHUNK) * CHUNK,
                                         rem), :])
            if with_counts:
                pltpu.sync_copy(zcb, cacc.at[pl.ds(base, 3128)])
                pltpu.sync_copy(zcb, cacc.at[pl.ds(base + 3128, 3128)])

        def count_cond(ci):
            return jnp.equal(ci < NCHUNK // 2, c == 0)

        def run_pipeline(y_hbm):
            def idx_row0(ci):
                return t * (NCHUNK * RPC) + ci * RPC

            def fire_idx(b, ci, sync=False):
                copy = pltpu.sync_copy if sync else (
                    lambda s, d: pltpu.async_copy(s, d, psem))
                copy(src_hbm.at[pl.ds(idx_row0(ci), RPC), :], src_v.at[b])
                copy(dst_hbm.at[pl.ds(idx_row0(ci), RPC), :], dst_v.at[b])

            def wait_idx(b, ci):
                pltpu.make_async_copy(
                    src_hbm.at[pl.ds(idx_row0(ci), RPC), :],
                    src_v.at[b], psem).wait()
                pltpu.make_async_copy(
                    dst_hbm.at[pl.ds(idx_row0(ci), RPC), :],
                    dst_v.at[b], psem).wait()

            def fire_gathers(b):
                for j in range(RPC):
                    pltpu.async_copy(y_hbm.at[src_v.at[b, j]],
                                     rows_v.at[b, pl.ds(j * 128, 128), :],
                                     gsem)

            def wait_gathers(b):
                for j in range(RPC):
                    pltpu.make_async_copy(
                        y_hbm.at[src_v.at[b, j]],
                        rows_v.at[b, pl.ds(j * 128, 128), :], gsem).wait()

            def fire_scatters(b, ci):
                for j in range(RPC):
                    pltpu.async_copy(rows_v.at[b, pl.ds(j * 128, 128), :],
                                     acc.at[dst_v.at[b, j]], ssem, add=True)
                if with_counts:
                    @pl.when(count_cond(ci))
                    def _():
                        for j in range(RPC):
                            pltpu.async_copy(ones_v, cacc.at[dst_v.at[b, j]],
                                             ssem, add=True)

            def wait_scatters(b, ci):
                for j in range(RPC):
                    pltpu.make_async_copy(rows_v.at[b, pl.ds(j * 128, 128), :],
                                          acc.at[dst_v.at[b, j]], ssem).wait()
                if with_counts:
                    @pl.when(count_cond(ci))
                    def _():
                        for j in range(RPC):
                            pltpu.make_async_copy(
                                ones_v, cacc.at[dst_v.at[b, j]], ssem).wait()

            fire_idx(0, 0, sync=True)
            fire_gathers(0)

            def pair_body(g, _):
                for b in (0, 1):
                    ci = 2 * g + b

                    @pl.when(ci >= 1)
                    def _():
                        wait_scatters(1 - b, ci - 1)

                    @pl.when(ci <= NCHUNK - 2)
                    def _():
                        fire_idx(1 - b, ci + 1)

                    wait_gathers(b)
                    fire_scatters(b, ci)

                    @pl.when(ci <= NCHUNK - 2)
                    def _():
                        wait_idx(1 - b, ci + 1)
                        fire_gathers(1 - b)
                return ()

            lax.fori_loop(0, NCHUNK // 2, pair_body, ())
            wait_scatters(1, NCHUNK - 1)

        def writeback():
            def wb_one(out_ref, cnt_ref):
                @pl.when(t < NS - 1)
                def _():
                    pltpu.sync_copy(acc.at[pl.ds(t * WB, WB), :],
                                    out_ref.at[pl.ds(t * WB, WB), :])
                    if with_counts:
                        pltpu.sync_copy(cacc.at[pl.ds(t * WB, WB)],
                                        cnt_ref.at[pl.ds(t * WB, WB)])

                @pl.when(t == NS - 1)
                def _():
                    pltpu.sync_copy(
                        acc.at[pl.ds((NS - 1) * WB, WB_LAST), :],
                        out_ref.at[pl.ds((NS - 1) * WB, WB_LAST), :])
                    if with_counts:
                        pltpu.sync_copy(
                            cacc.at[pl.ds((NS - 1) * WB, WB_LAST)],
                            cnt_ref.at[pl.ds((NS - 1) * WB, WB_LAST)])

            @pl.when(c == 0)
            def _():
                wb_one(out_lo, cnt_lo)

            @pl.when(c == 1)
            def _():
                wb_one(out_hi, cnt_hi)

        zero_acc()
        plsc.subcore_barrier()

        @pl.when(c == 0)
        def _():
            run_pipeline(y_lo)

        @pl.when(c == 1)
        def _():
            run_pipeline(y_hi)

        plsc.subcore_barrier()
        writeback()

    return seg_kernel


_seg1_kernel = _make_seg_kernel(True)
_seg2_kernel = _make_seg_kernel(False)


def _dotT(x, w):
    return lax.dot_general(x, w, (((1,), (1,)), ((), ())),
                           preferred_element_type=jnp.float32)


BR = 2000


def _tc0_body(xu, xm, wl1m, wr1m, b1m, wl1u, wr1u, b1u,
              yu_lo, yu_hi, ym_lo, ym_hi, rm, ru):
    yu = _dotT(xu[...], wl1m[...])
    yu_lo[...] = yu[:, :HH]
    yu_hi[...] = yu[:, HH:]
    ym = _dotT(xm[...], wl1u[...])
    ym_lo[...] = ym[:, :HH]
    ym_hi[...] = ym[:, HH:]
    rm[...] = _dotT(xm[...], wr1m[...]) + b1m[...]
    ru[...] = _dotT(xu[...], wr1u[...]) + b1u[...]


def _tc2_body(s1m_lo, s1m_hi, cm0, cm1, rm, s1u_lo, s1u_hi, cu0, cu1, ru,
              wl2m, wr2m, b2m, wl2u, wr2u, b2u,
              zu_lo, zu_hi, zm_lo, zm_hi, r2m, r2u):
    inv_cm = 1.0 / jnp.clip(cm0[...] + cm1[...], 1.0)
    inv_cu = 1.0 / jnp.clip(cu0[...] + cu1[...], 1.0)
    m1 = jnp.concatenate([s1m_lo[...], s1m_hi[...]], axis=1) * inv_cm + rm[...]
    u1 = jnp.concatenate([s1u_lo[...], s1u_hi[...]], axis=1) * inv_cu + ru[...]
    zu = _dotT(u1, wl2m[...])
    zu_lo[...] = zu[:, :HH]
    zu_hi[...] = zu[:, HH:]
    zm = _dotT(m1, wl2u[...])
    zm_lo[...] = zm[:, :HH]
    zm_hi[...] = zm[:, HH:]
    r2m[...] = _dotT(m1, wr2m[...]) + b2m[...]
    r2u[...] = _dotT(u1, wr2u[...]) + b2u[...]


def _tc4_body(s2m_lo, s2m_hi, cm0, cm1, r2m, s2u_lo, s2u_hi, cu0, cu1, r2u,
              u2, m2):
    inv_cm = 1.0 / jnp.clip(cm0[...] + cm1[...], 1.0)
    inv_cu = 1.0 / jnp.clip(cu0[...] + cu1[...], 1.0)
    m2[...] = (jnp.concatenate([s2m_lo[...], s2m_hi[...]], axis=1)
               * inv_cm + r2m[...])
    u2[...] = (jnp.concatenate([s2u_lo[...], s2u_hi[...]], axis=1)
               * inv_cu + r2u[...])


def _rows(nc):
    return pl.BlockSpec((BR, nc), lambda i: (i, 0))


def _full(shape):
    nd = len(shape)
    return pl.BlockSpec(shape, lambda i: (0,) * nd)


def _shape(r, c):
    return jax.ShapeDtypeStruct((r, c), jnp.float32)


def kernel(x_user, x_movie, edge_index_rates, edge_index_rated_by,
           Wl1m, b1m, Wr1m, Wl1u, b1u, Wr1u,
           Wl2m, b2m, Wr2m, Wl2u, b2u, Wr2u):
    grid = (N // BR,)

    yu_lo, yu_hi, ym_lo, ym_hi, rm, ru = pl.pallas_call(
        _tc0_body,
        grid=grid,
        in_specs=[_rows(24), _rows(18),
                  _full((H, 24)), _full((H, 18)), _full((1, H)),
                  _full((H, 18)), _full((H, 24)), _full((1, H))],
        out_specs=[_rows(HH), _rows(HH), _rows(HH), _rows(HH),
                   _rows(H), _rows(H)],
        out_shape=[_shape(N, HH), _shape(N, HH), _shape(N, HH), _shape(N, HH),
                   _shape(N, H), _shape(N, H)],
    )(x_user, x_movie, Wl1m, Wr1m, b1m.reshape(1, H),
      Wl1u, Wr1u, b1u.reshape(1, H))

    pad = (jnp.arange(PAD, dtype=jnp.int32) % NPADROW)

    def prep(ei):
        src = jnp.concatenate([ei[0].astype(jnp.int32), pad])
        dst = jnp.concatenate([ei[1].astype(jnp.int32), pad + N])
        return src.reshape(EROWS, 128), dst.reshape(EROWS, 128)

    src_r, dst_r = prep(edge_index_rates)
    src_b, dst_b = prep(edge_index_rated_by)

    s1m_lo, s1m_hi, cm0, cm1 = _seg1_kernel(src_r, dst_r, yu_lo, yu_hi)
    s1u_lo, s1u_hi, cu0, cu1 = _seg1_kernel(src_b, dst_b, ym_lo, ym_hi)
    cm0, cm1 = cm0.reshape(N, 1), cm1.reshape(N, 1)
    cu0, cu1 = cu0.reshape(N, 1), cu1.reshape(N, 1)

    zu_lo, zu_hi, zm_lo, zm_hi, r2m, r2u = pl.pallas_call(
        _tc2_body,
        grid=grid,
        in_specs=[_rows(HH), _rows(HH), _rows(1), _rows(1), _rows(H),
                  _rows(HH), _rows(HH), _rows(1), _rows(1), _rows(H),
                  _full((H, H)), _full((H, H)), _full((1, H)),
                  _full((H, H)), _full((H, H)), _full((1, H))],
        out_specs=[_rows(HH), _rows(HH), _rows(HH), _rows(HH),
                   _rows(H), _rows(H)],
        out_shape=[_shape(N, HH), _shape(N, HH), _shape(N, HH), _shape(N, HH),
                   _shape(N, H), _shape(N, H)],
    )(s1m_lo, s1m_hi, cm0, cm1, rm, s1u_lo, s1u_hi, cu0, cu1, ru,
      Wl2m, Wr2m, b2m.reshape(1, H), Wl2u, Wr2u, b2u.reshape(1, H))

    s2m_lo, s2m_hi = _seg2_kernel(src_r, dst_r, zu_lo, zu_hi)
    s2u_lo, s2u_hi = _seg2_kernel(src_b, dst_b, zm_lo, zm_hi)

    u2, m2 = pl.pallas_call(
        _tc4_body,
        grid=grid,
        in_specs=[_rows(HH), _rows(HH), _rows(1), _rows(1), _rows(H),
                  _rows(HH), _rows(HH), _rows(1), _rows(1), _rows(H)],
        out_specs=[_rows(H), _rows(H)],
        out_shape=[_shape(N, H), _shape(N, H)],
    )(s2m_lo, s2m_hi, cm0, cm1, r2m, s2u_lo, s2u_hi, cu0, cu1, r2u)

    return (u2, m2)


# speedup vs baseline: 1.1853x; 1.1004x over previous
import functools

import jax
import jax.numpy as jnp
from jax import lax
from jax.experimental import pallas as pl
from jax.experimental.pallas import tpu as pltpu
from jax.experimental.pallas import tpu_sc as plsc

N = 100000
E = 1600000
H = 32
HH = 16
NC = 2
NS = 16

CHUNK = 512
RPC = CHUNK // 128
NCHUNK = 200
EPT = CHUNK * NCHUNK
E_PAD = EPT * NS
EROWS = E_PAD // 128
PAD = E_PAD - E
NPADROW = 2144
R = N + NPADROW
ZPT = R // NS
WB = 6256
WB_LAST = N - 15 * WB

CRPC = 16
CNCH = EROWS // NS // CRPC

_MESH = dict(core_axis_name="c", subcore_axis_name="s", num_cores=NC,
             num_subcores=NS)
_SC_PARAMS = pltpu.CompilerParams(use_tc_tiling_on_sc=False)


def _zero_rows(rows_v, nrows):
    zero = jnp.zeros((16,), jnp.float32)

    def body(i, _):
        rows_v[i, :] = zero
        return ()

    lax.fori_loop(0, nrows, body, ())


@functools.partial(
    pl.kernel,
    out_type=[jax.ShapeDtypeStruct((N, HH), jnp.float32),
              jax.ShapeDtypeStruct((N, HH), jnp.float32)],
    mesh=plsc.VectorSubcoreMesh(**_MESH),
    scratch_types=[
        pltpu.VMEM((2, RPC, 128), jnp.int32),
        pltpu.VMEM((2, RPC, 128), jnp.int32),
        pltpu.VMEM((2, CHUNK, HH), jnp.float32),
        pltpu.VMEM_SHARED((R, HH), jnp.float32),
        pltpu.SemaphoreType.DMA,
        pltpu.SemaphoreType.DMA,
        pltpu.SemaphoreType.DMA,
    ],
    compiler_params=_SC_PARAMS,
)
def _seg_kernel(src_hbm, dst_hbm, y_lo, y_hi, out_lo, out_hi,
                src_v, dst_v, rows_v, acc, gsem, ssem, psem):
    c = lax.axis_index("c")
    t = lax.axis_index("s")

    _zero_rows(rows_v.at[0], CHUNK)
    base = t * ZPT
    for j in range(ZPT // CHUNK):
        pltpu.sync_copy(rows_v.at[0], acc.at[pl.ds(base + j * CHUNK, CHUNK), :])
    rem = ZPT % CHUNK
    pltpu.sync_copy(rows_v.at[0, pl.ds(0, rem), :],
                    acc.at[pl.ds(base + (ZPT // CHUNK) * CHUNK, rem), :])
    plsc.subcore_barrier()

    def run_core(y_hbm):
        def idx_row0(ci):
            return t * (NCHUNK * RPC) + ci * RPC

        def fire_idx(b, ci, sync=False):
            copy = pltpu.sync_copy if sync else (
                lambda s, d: pltpu.async_copy(s, d, psem))
            copy(src_hbm.at[pl.ds(idx_row0(ci), RPC), :], src_v.at[b])
            copy(dst_hbm.at[pl.ds(idx_row0(ci), RPC), :], dst_v.at[b])

        def wait_idx(b, ci):
            pltpu.make_async_copy(src_hbm.at[pl.ds(idx_row0(ci), RPC), :],
                                  src_v.at[b], psem).wait()
            pltpu.make_async_copy(dst_hbm.at[pl.ds(idx_row0(ci), RPC), :],
                                  dst_v.at[b], psem).wait()

        def fire_gathers(b):
            for j in range(RPC):
                pltpu.async_copy(y_hbm.at[src_v.at[b, j]],
                                 rows_v.at[b, pl.ds(j * 128, 128), :], gsem)

        def wait_gathers(b):
            for j in range(RPC):
                pltpu.make_async_copy(y_hbm.at[src_v.at[b, j]],
                                      rows_v.at[b, pl.ds(j * 128, 128), :],
                                      gsem).wait()

        def fire_scatters(b):
            for j in range(RPC):
                pltpu.async_copy(rows_v.at[b, pl.ds(j * 128, 128), :],
                                 acc.at[dst_v.at[b, j]], ssem, add=True)

        def wait_scatters(b):
            for j in range(RPC):
                pltpu.make_async_copy(rows_v.at[b, pl.ds(j * 128, 128), :],
                                      acc.at[dst_v.at[b, j]], ssem).wait()

        fire_idx(0, 0, sync=True)
        fire_gathers(0)

        def pair_body(g, _):
            for b in (0, 1):
                ci = 2 * g + b

                @pl.when(ci >= 1)
                def _():
                    wait_scatters(1 - b)

                @pl.when(ci <= NCHUNK - 2)
                def _():
                    fire_idx(1 - b, ci + 1)

                wait_gathers(b)
                fire_scatters(b)

                @pl.when(ci <= NCHUNK - 2)
                def _():
                    wait_idx(1 - b, ci + 1)
                    fire_gathers(1 - b)
            return ()

        lax.fori_loop(0, NCHUNK // 2, pair_body, ())
        wait_scatters(1)

    @pl.when(c == 0)
    def _():
        run_core(y_lo)

    @pl.when(c == 1)
    def _():
        run_core(y_hi)

    plsc.subcore_barrier()

    def writeback(out_ref):
        @pl.when(t < NS - 1)
        def _():
            pltpu.sync_copy(acc.at[pl.ds(t * WB, WB), :],
                            out_ref.at[pl.ds(t * WB, WB), :])

        @pl.when(t == NS - 1)
        def _():
            pltpu.sync_copy(acc.at[pl.ds((NS - 1) * WB, WB_LAST), :],
                            out_ref.at[pl.ds((NS - 1) * WB, WB_LAST), :])

    @pl.when(c == 0)
    def _():
        writeback(out_lo)

    @pl.when(c == 1)
    def _():
        writeback(out_hi)


@functools.partial(
    pl.kernel,
    out_type=[jax.ShapeDtypeStruct((N,), jnp.float32),
              jax.ShapeDtypeStruct((N,), jnp.float32)],
    mesh=plsc.VectorSubcoreMesh(**_MESH),
    scratch_types=[
        pltpu.VMEM((2, CRPC, 128), jnp.int32),
        pltpu.VMEM((128,), jnp.float32),
        pltpu.VMEM((ZPT,), jnp.float32),
        pltpu.VMEM_SHARED((R,), jnp.float32),
        pltpu.SemaphoreType.DMA,
        pltpu.SemaphoreType.DMA,
    ],
    compiler_params=_SC_PARAMS,
)
def _cnt_kernel(dst_r_hbm, dst_b_hbm, cnt_r, cnt_b,
                dst_v, ones_v, zcb, cacc, ssem, psem):
    c = lax.axis_index("c")
    t = lax.axis_index("s")

    for j in range(ZPT // 16):
        zcb[pl.ds(j * 16, 16)] = jnp.zeros((16,), jnp.float32)
    for j in range(8):
        ones_v[pl.ds(j * 16, 16)] = jnp.ones((16,), jnp.float32)
    pltpu.sync_copy(zcb, cacc.at[pl.ds(t * ZPT, ZPT)])
    plsc.subcore_barrier()

    def run_core(dst_hbm):
        def idx_row0(ci):
            return t * (CNCH * CRPC) + ci * CRPC

        def fire_idx(b, ci, sync=False):
            copy = pltpu.sync_copy if sync else (
                lambda s, d: pltpu.async_copy(s, d, psem))
            copy(dst_hbm.at[pl.ds(idx_row0(ci), CRPC), :], dst_v.at[b])

        def wait_idx(b, ci):
            pltpu.make_async_copy(dst_hbm.at[pl.ds(idx_row0(ci), CRPC), :],
                                  dst_v.at[b], psem).wait()

        def fire_scatters(b):
            for j in range(CRPC):
                pltpu.async_copy(ones_v, cacc.at[dst_v.at[b, j]], ssem,
                                 add=True)

        def wait_scatters(b):
            for j in range(CRPC):
                pltpu.make_async_copy(ones_v, cacc.at[dst_v.at[b, j]],
                                      ssem).wait()

        fire_idx(0, 0, sync=True)

        def pair_body(g, _):
            for b in (0, 1):
                ci = 2 * g + b

                @pl.when(ci >= 1)
                def _():
                    wait_scatters(1 - b)

                @pl.when(ci <= CNCH - 2)
                def _():
                    fire_idx(1 - b, ci + 1)

                @pl.when(ci >= 1)
                def _():
                    wait_idx(b, ci)

                fire_scatters(b)
            return ()

        lax.fori_loop(0, CNCH // 2, pair_body, ())
        wait_scatters(1)

    @pl.when(c == 0)
    def _():
        run_core(dst_r_hbm)

    @pl.when(c == 1)
    def _():
        run_core(dst_b_hbm)

    plsc.subcore_barrier()

    def writeback(cnt_ref):
        @pl.when(t < NS - 1)
        def _():
            pltpu.sync_copy(cacc.at[pl.ds(t * WB, WB)],
                            cnt_ref.at[pl.ds(t * WB, WB)])

        @pl.when(t == NS - 1)
        def _():
            pltpu.sync_copy(cacc.at[pl.ds((NS - 1) * WB, WB_LAST)],
                            cnt_ref.at[pl.ds((NS - 1) * WB, WB_LAST)])

    @pl.when(c == 0)
    def _():
        writeback(cnt_r)

    @pl.when(c == 1)
    def _():
        writeback(cnt_b)


def _dotT(x, w):
    return lax.dot_general(x, w, (((1,), (1,)), ((), ())),
                           preferred_element_type=jnp.float32)


BR = 2000


def _tc0_body(xu, xm, wl1m, wr1m, b1m, wl1u, wr1u, b1u,
              yu_lo, yu_hi, ym_lo, ym_hi, rm, ru):
    yu = _dotT(xu[...], wl1m[...])
    yu_lo[...] = yu[:, :HH]
    yu_hi[...] = yu[:, HH:]
    ym = _dotT(xm[...], wl1u[...])
    ym_lo[...] = ym[:, :HH]
    ym_hi[...] = ym[:, HH:]
    rm[...] = _dotT(xm[...], wr1m[...]) + b1m[...]
    ru[...] = _dotT(xu[...], wr1u[...]) + b1u[...]


def _tc2_body(s1m_lo, s1m_hi, cm, rm, s1u_lo, s1u_hi, cu, ru,
              wl2m, wr2m, b2m, wl2u, wr2u, b2u,
              zu_lo, zu_hi, zm_lo, zm_hi, r2m, r2u):
    inv_cm = 1.0 / jnp.clip(cm[...], 1.0)
    inv_cu = 1.0 / jnp.clip(cu[...], 1.0)
    m1 = jnp.concatenate([s1m_lo[...], s1m_hi[...]], axis=1) * inv_cm + rm[...]
    u1 = jnp.concatenate([s1u_lo[...], s1u_hi[...]], axis=1) * inv_cu + ru[...]
    zu = _dotT(u1, wl2m[...])
    zu_lo[...] = zu[:, :HH]
    zu_hi[...] = zu[:, HH:]
    zm = _dotT(m1, wl2u[...])
    zm_lo[...] = zm[:, :HH]
    zm_hi[...] = zm[:, HH:]
    r2m[...] = _dotT(m1, wr2m[...]) + b2m[...]
    r2u[...] = _dotT(u1, wr2u[...]) + b2u[...]


def _tc4_body(s2m_lo, s2m_hi, cm, r2m, s2u_lo, s2u_hi, cu, r2u,
              u2, m2):
    inv_cm = 1.0 / jnp.clip(cm[...], 1.0)
    inv_cu = 1.0 / jnp.clip(cu[...], 1.0)
    m2[...] = (jnp.concatenate([s2m_lo[...], s2m_hi[...]], axis=1)
               * inv_cm + r2m[...])
    u2[...] = (jnp.concatenate([s2u_lo[...], s2u_hi[...]], axis=1)
               * inv_cu + r2u[...])


def _rows(nc):
    return pl.BlockSpec((BR, nc), lambda i: (i, 0))


def _full(shape):
    nd = len(shape)
    return pl.BlockSpec(shape, lambda i: (0,) * nd)


def _shape(r, c):
    return jax.ShapeDtypeStruct((r, c), jnp.float32)


def kernel(x_user, x_movie, edge_index_rates, edge_index_rated_by,
           Wl1m, b1m, Wr1m, Wl1u, b1u, Wr1u,
           Wl2m, b2m, Wr2m, Wl2u, b2u, Wr2u):
    grid = (N // BR,)

    yu_lo, yu_hi, ym_lo, ym_hi, rm, ru = pl.pallas_call(
        _tc0_body,
        grid=grid,
        in_specs=[_rows(24), _rows(18),
                  _full((H, 24)), _full((H, 18)), _full((1, H)),
                  _full((H, 18)), _full((H, 24)), _full((1, H))],
        out_specs=[_rows(HH), _rows(HH), _rows(HH), _rows(HH),
                   _rows(H), _rows(H)],
        out_shape=[_shape(N, HH), _shape(N, HH), _shape(N, HH), _shape(N, HH),
                   _shape(N, H), _shape(N, H)],
    )(x_user, x_movie, Wl1m, Wr1m, b1m.reshape(1, H),
      Wl1u, Wr1u, b1u.reshape(1, H))

    pad = (jnp.arange(PAD, dtype=jnp.int32) % NPADROW)

    def prep(ei):
        src = jnp.concatenate([ei[0].astype(jnp.int32), pad])
        dst = jnp.concatenate([ei[1].astype(jnp.int32), pad + N])
        return src.reshape(EROWS, 128), dst.reshape(EROWS, 128)

    src_r, dst_r = prep(edge_index_rates)
    src_b, dst_b = prep(edge_index_rated_by)

    cnt_m, cnt_u = _cnt_kernel(dst_r, dst_b)
    s1m_lo, s1m_hi = _seg_kernel(src_r, dst_r, yu_lo, yu_hi)
    s1u_lo, s1u_hi = _seg_kernel(src_b, dst_b, ym_lo, ym_hi)
    cm = cnt_m.reshape(N, 1)
    cu = cnt_u.reshape(N, 1)

    zu_lo, zu_hi, zm_lo, zm_hi, r2m, r2u = pl.pallas_call(
        _tc2_body,
        grid=grid,
        in_specs=[_rows(HH), _rows(HH), _rows(1), _rows(H),
                  _rows(HH), _rows(HH), _rows(1), _rows(H),
                  _full((H, H)), _full((H, H)), _full((1, H)),
                  _full((H, H)), _full((H, H)), _full((1, H))],
        out_specs=[_rows(HH), _rows(HH), _rows(HH), _rows(HH),
                   _rows(H), _rows(H)],
        out_shape=[_shape(N, HH), _shape(N, HH), _shape(N, HH), _shape(N, HH),
                   _shape(N, H), _shape(N, H)],
    )(s1m_lo, s1m_hi, cm, rm, s1u_lo, s1u_hi, cu, ru,
      Wl2m, Wr2m, b2m.reshape(1, H), Wl2u, Wr2u, b2u.reshape(1, H))

    s2m_lo, s2m_hi = _seg_kernel(src_r, dst_r, zu_lo, zu_hi)
    s2u_lo, s2u_hi = _seg_kernel(src_b, dst_b, zm_lo, zm_hi)

    u2, m2 = pl.pallas_call(
        _tc4_body,
        grid=grid,
        in_specs=[_rows(HH), _rows(HH), _rows(1), _rows(H),
                  _rows(HH), _rows(HH), _rows(1), _rows(H)],
        out_specs=[_rows(H), _rows(H)],
        out_shape=[_shape(N, H), _shape(N, H)],
    )(s2m_lo, s2m_hi, cm, r2m, s2u_lo, s2u_hi, cu, r2u)

    return (u2, m2)


# speedup vs baseline: 1.2663x; 1.0683x over previous
import functools

import jax
import jax.numpy as jnp
from jax import lax
from jax.experimental import pallas as pl
from jax.experimental.pallas import tpu as pltpu
from jax.experimental.pallas import tpu_sc as plsc

N = 100000
E = 1600000
H = 32
HH = 16
NC = 2
NS = 16

CHUNK = 768
RPC = CHUNK // 128
NCHUNK = 136
EPT = CHUNK * NCHUNK
E_PAD = EPT * NS
EROWS = E_PAD // 128
PAD = E_PAD - E
NPADROW = 2144
R = N + NPADROW
ZPT = R // NS
WB = 6256
WB_LAST = N - 15 * WB

CRPC = 24
CNCH = EROWS // NS // CRPC

_MESH = dict(core_axis_name="c", subcore_axis_name="s", num_cores=NC,
             num_subcores=NS)
_SC_PARAMS = pltpu.CompilerParams(use_tc_tiling_on_sc=False)


def _zero_rows(rows_v, nrows):
    zero = jnp.zeros((16,), jnp.float32)

    def body(i, _):
        rows_v[i, :] = zero
        return ()

    lax.fori_loop(0, nrows, body, ())


@functools.partial(
    pl.kernel,
    out_type=[jax.ShapeDtypeStruct((N, HH), jnp.float32),
              jax.ShapeDtypeStruct((N, HH), jnp.float32)],
    mesh=plsc.VectorSubcoreMesh(**_MESH),
    scratch_types=[
        pltpu.VMEM((2, RPC, 128), jnp.int32),
        pltpu.VMEM((2, RPC, 128), jnp.int32),
        pltpu.VMEM((2, CHUNK, HH), jnp.float32),
        pltpu.VMEM_SHARED((R, HH), jnp.float32),
        pltpu.SemaphoreType.DMA,
        pltpu.SemaphoreType.DMA,
        pltpu.SemaphoreType.DMA,
    ],
    compiler_params=_SC_PARAMS,
)
def _seg_kernel(src_hbm, dst_hbm, y_lo, y_hi, out_lo, out_hi,
                src_v, dst_v, rows_v, acc, gsem, ssem, psem):
    c = lax.axis_index("c")
    t = lax.axis_index("s")

    _zero_rows(rows_v.at[0], CHUNK)
    base = t * ZPT
    for j in range(ZPT // CHUNK):
        pltpu.sync_copy(rows_v.at[0], acc.at[pl.ds(base + j * CHUNK, CHUNK), :])
    rem = ZPT % CHUNK
    pltpu.sync_copy(rows_v.at[0, pl.ds(0, rem), :],
                    acc.at[pl.ds(base + (ZPT // CHUNK) * CHUNK, rem), :])
    plsc.subcore_barrier()

    def run_core(y_hbm):
        def idx_row0(ci):
            return t * (NCHUNK * RPC) + ci * RPC

        def fire_idx(b, ci, sync=False):
            copy = pltpu.sync_copy if sync else (
                lambda s, d: pltpu.async_copy(s, d, psem))
            copy(src_hbm.at[pl.ds(idx_row0(ci), RPC), :], src_v.at[b])
            copy(dst_hbm.at[pl.ds(idx_row0(ci), RPC), :], dst_v.at[b])

        def wait_idx(b, ci):
            pltpu.make_async_copy(src_hbm.at[pl.ds(idx_row0(ci), RPC), :],
                                  src_v.at[b], psem).wait()
            pltpu.make_async_copy(dst_hbm.at[pl.ds(idx_row0(ci), RPC), :],
                                  dst_v.at[b], psem).wait()

        def fire_gathers(b):
            for j in range(RPC):
                pltpu.async_copy(y_hbm.at[src_v.at[b, j]],
                                 rows_v.at[b, pl.ds(j * 128, 128), :], gsem)

        def wait_gathers(b):
            for j in range(RPC):
                pltpu.make_async_copy(y_hbm.at[src_v.at[b, j]],
                                      rows_v.at[b, pl.ds(j * 128, 128), :],
                                      gsem).wait()

        def fire_scatters(b):
            for j in range(RPC):
                pltpu.async_copy(rows_v.at[b, pl.ds(j * 128, 128), :],
                                 acc.at[dst_v.at[b, j]], ssem, add=True)

        def wait_scatters(b):
            for j in range(RPC):
                pltpu.make_async_copy(rows_v.at[b, pl.ds(j * 128, 128), :],
                                      acc.at[dst_v.at[b, j]], ssem).wait()

        fire_idx(0, 0, sync=True)
        fire_gathers(0)

        def pair_body(g, _):
            for b in (0, 1):
                ci = 2 * g + b

                @pl.when(ci >= 1)
                def _():
                    wait_scatters(1 - b)

                @pl.when(ci <= NCHUNK - 2)
                def _():
                    fire_idx(1 - b, ci + 1)

                wait_gathers(b)
                fire_scatters(b)

                @pl.when(ci <= NCHUNK - 2)
                def _():
                    wait_idx(1 - b, ci + 1)
                    fire_gathers(1 - b)
            return ()

        lax.fori_loop(0, NCHUNK // 2, pair_body, ())
        wait_scatters(1)

    @pl.when(c == 0)
    def _():
        run_core(y_lo)

    @pl.when(c == 1)
    def _():
        run_core(y_hi)

    plsc.subcore_barrier()

    def writeback(out_ref):
        @pl.when(t < NS - 1)
        def _():
            pltpu.sync_copy(acc.at[pl.ds(t * WB, WB), :],
                            out_ref.at[pl.ds(t * WB, WB), :])

        @pl.when(t == NS - 1)
        def _():
            pltpu.sync_copy(acc.at[pl.ds((NS - 1) * WB, WB_LAST), :],
                            out_ref.at[pl.ds((NS - 1) * WB, WB_LAST), :])

    @pl.when(c == 0)
    def _():
        writeback(out_lo)

    @pl.when(c == 1)
    def _():
        writeback(out_hi)


@functools.partial(
    pl.kernel,
    out_type=[jax.ShapeDtypeStruct((N,), jnp.float32),
              jax.ShapeDtypeStruct((N,), jnp.float32)],
    mesh=plsc.VectorSubcoreMesh(**_MESH),
    scratch_types=[
        pltpu.VMEM((2, CRPC, 128), jnp.int32),
        pltpu.VMEM((128,), jnp.float32),
        pltpu.VMEM((ZPT,), jnp.float32),
        pltpu.VMEM_SHARED((R,), jnp.float32),
        pltpu.SemaphoreType.DMA,
        pltpu.SemaphoreType.DMA,
    ],
    compiler_params=_SC_PARAMS,
)
def _cnt_kernel(dst_r_hbm, dst_b_hbm, cnt_r, cnt_b,
                dst_v, ones_v, zcb, cacc, ssem, psem):
    c = lax.axis_index("c")
    t = lax.axis_index("s")

    for j in range(ZPT // 16):
        zcb[pl.ds(j * 16, 16)] = jnp.zeros((16,), jnp.float32)
    for j in range(8):
        ones_v[pl.ds(j * 16, 16)] = jnp.ones((16,), jnp.float32)
    pltpu.sync_copy(zcb, cacc.at[pl.ds(t * ZPT, ZPT)])
    plsc.subcore_barrier()

    def run_core(dst_hbm):
        def idx_row0(ci):
            return t * (CNCH * CRPC) + ci * CRPC

        def fire_idx(b, ci, sync=False):
            copy = pltpu.sync_copy if sync else (
                lambda s, d: pltpu.async_copy(s, d, psem))
            copy(dst_hbm.at[pl.ds(idx_row0(ci), CRPC), :], dst_v.at[b])

        def wait_idx(b, ci):
            pltpu.make_async_copy(dst_hbm.at[pl.ds(idx_row0(ci), CRPC), :],
                                  dst_v.at[b], psem).wait()

        def fire_scatters(b):
            for j in range(CRPC):
                pltpu.async_copy(ones_v, cacc.at[dst_v.at[b, j]], ssem,
                                 add=True)

        def wait_scatters(b):
            for j in range(CRPC):
                pltpu.make_async_copy(ones_v, cacc.at[dst_v.at[b, j]],
                                      ssem).wait()

        fire_idx(0, 0, sync=True)

        def pair_body(g, _):
            for b in (0, 1):
                ci = 2 * g + b

                @pl.when(ci >= 1)
                def _():
                    wait_scatters(1 - b)

                @pl.when(ci <= CNCH - 2)
                def _():
                    fire_idx(1 - b, ci + 1)

                @pl.when(ci >= 1)
                def _():
                    wait_idx(b, ci)

                fire_scatters(b)
            return ()

        lax.fori_loop(0, CNCH // 2, pair_body, ())
        wait_scatters(1)

    @pl.when(c == 0)
    def _():
        run_core(dst_r_hbm)

    @pl.when(c == 1)
    def _():
        run_core(dst_b_hbm)

    plsc.subcore_barrier()

    def writeback(cnt_ref):
        @pl.when(t < NS - 1)
        def _():
            pltpu.sync_copy(cacc.at[pl.ds(t * WB, WB)],
                            cnt_ref.at[pl.ds(t * WB, WB)])

        @pl.when(t == NS - 1)
        def _():
            pltpu.sync_copy(cacc.at[pl.ds((NS - 1) * WB, WB_LAST)],
                            cnt_ref.at[pl.ds((NS - 1) * WB, WB_LAST)])

    @pl.when(c == 0)
    def _():
        writeback(cnt_r)

    @pl.when(c == 1)
    def _():
        writeback(cnt_b)


def _dotT(x, w):
    return lax.dot_general(x, w, (((1,), (1,)), ((), ())),
                           preferred_element_type=jnp.float32)


BR = 2000


def _tc0_body(xu, xm, wl1m, wr1m, b1m, wl1u, wr1u, b1u,
              yu_lo, yu_hi, ym_lo, ym_hi, rm, ru):
    yu = _dotT(xu[...], wl1m[...])
    yu_lo[...] = yu[:, :HH]
    yu_hi[...] = yu[:, HH:]
    ym = _dotT(xm[...], wl1u[...])
    ym_lo[...] = ym[:, :HH]
    ym_hi[...] = ym[:, HH:]
    rm[...] = _dotT(xm[...], wr1m[...]) + b1m[...]
    ru[...] = _dotT(xu[...], wr1u[...]) + b1u[...]


def _tc2_body(s1m_lo, s1m_hi, cm, rm, s1u_lo, s1u_hi, cu, ru,
              wl2m, wr2m, b2m, wl2u, wr2u, b2u,
              zu_lo, zu_hi, zm_lo, zm_hi, r2m, r2u):
    inv_cm = 1.0 / jnp.clip(cm[...], 1.0)
    inv_cu = 1.0 / jnp.clip(cu[...], 1.0)
    m1 = jnp.concatenate([s1m_lo[...], s1m_hi[...]], axis=1) * inv_cm + rm[...]
    u1 = jnp.concatenate([s1u_lo[...], s1u_hi[...]], axis=1) * inv_cu + ru[...]
    zu = _dotT(u1, wl2m[...])
    zu_lo[...] = zu[:, :HH]
    zu_hi[...] = zu[:, HH:]
    zm = _dotT(m1, wl2u[...])
    zm_lo[...] = zm[:, :HH]
    zm_hi[...] = zm[:, HH:]
    r2m[...] = _dotT(m1, wr2m[...]) + b2m[...]
    r2u[...] = _dotT(u1, wr2u[...]) + b2u[...]


def _tc4_body(s2m_lo, s2m_hi, cm, r2m, s2u_lo, s2u_hi, cu, r2u,
              u2, m2):
    inv_cm = 1.0 / jnp.clip(cm[...], 1.0)
    inv_cu = 1.0 / jnp.clip(cu[...], 1.0)
    m2[...] = (jnp.concatenate([s2m_lo[...], s2m_hi[...]], axis=1)
               * inv_cm + r2m[...])
    u2[...] = (jnp.concatenate([s2u_lo[...], s2u_hi[...]], axis=1)
               * inv_cu + r2u[...])


def _rows(nc):
    return pl.BlockSpec((BR, nc), lambda i: (i, 0))


def _full(shape):
    nd = len(shape)
    return pl.BlockSpec(shape, lambda i: (0,) * nd)


def _shape(r, c):
    return jax.ShapeDtypeStruct((r, c), jnp.float32)


def kernel(x_user, x_movie, edge_index_rates, edge_index_rated_by,
           Wl1m, b1m, Wr1m, Wl1u, b1u, Wr1u,
           Wl2m, b2m, Wr2m, Wl2u, b2u, Wr2u):
    grid = (N // BR,)

    yu_lo, yu_hi, ym_lo, ym_hi, rm, ru = pl.pallas_call(
        _tc0_body,
        grid=grid,
        in_specs=[_rows(24), _rows(18),
                  _full((H, 24)), _full((H, 18)), _full((1, H)),
                  _full((H, 18)), _full((H, 24)), _full((1, H))],
        out_specs=[_rows(HH), _rows(HH), _rows(HH), _rows(HH),
                   _rows(H), _rows(H)],
        out_shape=[_shape(N, HH), _shape(N, HH), _shape(N, HH), _shape(N, HH),
                   _shape(N, H), _shape(N, H)],
    )(x_user, x_movie, Wl1m, Wr1m, b1m.reshape(1, H),
      Wl1u, Wr1u, b1u.reshape(1, H))

    pad = (jnp.arange(PAD, dtype=jnp.int32) % NPADROW)

    def prep(ei):
        src = jnp.concatenate([ei[0].astype(jnp.int32), pad])
        dst = jnp.concatenate([ei[1].astype(jnp.int32), pad + N])
        return src.reshape(EROWS, 128), dst.reshape(EROWS, 128)

    src_r, dst_r = prep(edge_index_rates)
    src_b, dst_b = prep(edge_index_rated_by)

    cnt_m, cnt_u = _cnt_kernel(dst_r, dst_b)
    s1m_lo, s1m_hi = _seg_kernel(src_r, dst_r, yu_lo, yu_hi)
    s1u_lo, s1u_hi = _seg_kernel(src_b, dst_b, ym_lo, ym_hi)
    cm = cnt_m.reshape(N, 1)
    cu = cnt_u.reshape(N, 1)

    zu_lo, zu_hi, zm_lo, zm_hi, r2m, r2u = pl.pallas_call(
        _tc2_body,
        grid=grid,
        in_specs=[_rows(HH), _rows(HH), _rows(1), _rows(H),
                  _rows(HH), _rows(HH), _rows(1), _rows(H),
                  _full((H, H)), _full((H, H)), _full((1, H)),
                  _full((H, H)), _full((H, H)), _full((1, H))],
        out_specs=[_rows(HH), _rows(HH), _rows(HH), _rows(HH),
                   _rows(H), _rows(H)],
        out_shape=[_shape(N, HH), _shape(N, HH), _shape(N, HH), _shape(N, HH),
                   _shape(N, H), _shape(N, H)],
    )(s1m_lo, s1m_hi, cm, rm, s1u_lo, s1u_hi, cu, ru,
      Wl2m, Wr2m, b2m.reshape(1, H), Wl2u, Wr2u, b2u.reshape(1, H))

    s2m_lo, s2m_hi = _seg_kernel(src_r, dst_r, zu_lo, zu_hi)
    s2u_lo, s2u_hi = _seg_kernel(src_b, dst_b, zm_lo, zm_hi)

    u2, m2 = pl.pallas_call(
        _tc4_body,
        grid=grid,
        in_specs=[_rows(HH), _rows(HH), _rows(1), _rows(H),
                  _rows(HH), _rows(HH), _rows(1), _rows(H)],
        out_specs=[_rows(H), _rows(H)],
        out_shape=[_shape(N, H), _shape(N, H)],
    )(s2m_lo, s2m_hi, cm, r2m, s2u_lo, s2u_hi, cu, r2u)

    return (u2, m2)
